# baseline (speedup 1.0000x reference)
import functools

import numpy as np
import jax
import jax.numpy as jnp
from jax import lax
from jax.experimental import pallas as pl
from jax.experimental.pallas import tpu as pltpu
from jax.experimental.pallas import tpu_sc as plsc

N = 10000
NPAD = 10240
E = 320000
PADW = 48
NC, NS = 2, 16
NW = NC * NS
EPW = E // NW
CHUNK = 200
NCHUNK = EPW // CHUNK

_pa = np.array([8 + 2 * i for i in range(8)] + [24 + 2 * i for i in range(8)],
               np.int32)
_PERM = np.concatenate([_pa, _pa + 1, np.arange(8, dtype=np.int32)])


def _prep_nodes_body(h_ref, wn_ref, ws_ref, y0_ref, hs_ref):
    h = h_ref[...]
    y0 = jnp.dot(h, wn_ref[...], preferred_element_type=jnp.float32)
    col = lax.broadcasted_iota(jnp.int32, y0.shape, 1)
    y0_ref[...] = jnp.where(col == 40, 1.0, y0)
    hs_ref[...] = jnp.dot(h, ws_ref[...], preferred_element_type=jnp.float32)


def _prep_nodes(h, wn0p, ws0p):
    B = 2000
    return pl.pallas_call(
        _prep_nodes_body,
        grid=(N // B,),
        in_specs=[pl.BlockSpec((B, 128), lambda i: (i, 0)),
                  pl.BlockSpec((128, PADW), lambda i: (0, 0)),
                  pl.BlockSpec((128, PADW), lambda i: (0, 0))],
        out_specs=[pl.BlockSpec((B, PADW), lambda i: (i, 0)),
                   pl.BlockSpec((B, PADW), lambda i: (i, 0))],
        out_shape=[jax.ShapeDtypeStruct((N, PADW), jnp.float32),
                   jax.ShapeDtypeStruct((N, PADW), jnp.float32)],
    )(h, wn0p, ws0p)


def _prep_trig_body(th_ref, cs_ref):
    t = th_ref[...]
    c1 = jnp.cos(t)
    s1 = jnp.sin(t)
    c2 = c1 * c1 - s1 * s1
    s2 = 2.0 * c1 * s1
    row = lax.broadcasted_iota(jnp.int32, (8, t.shape[1]), 0)
    cs_ref[...] = jnp.where(row == 0, c1,
                            jnp.where(row == 1, s1,
                                      jnp.where(row == 2, c2,
                                                jnp.where(row == 3, s2, 0.0))))


def _prep_trig(theta2d):
    B = 16000
    return pl.pallas_call(
        _prep_trig_body,
        grid=(E // B,),
        in_specs=[pl.BlockSpec((1, B), lambda i: (0, i))],
        out_specs=pl.BlockSpec((8, B), lambda i: (0, i)),
        out_shape=jax.ShapeDtypeStruct((8, E), jnp.float32),
    )(theta2d)


def _make_conv(rotate, interpret=False):
    mesh = plsc.VectorSubcoreMesh(core_axis_name="c", subcore_axis_name="s",
                                  num_cores=NC, num_subcores=NS)

    def body_impl(table_hbm, src_hbm, dst_hbm, cs_hbm, zero_hbm, out_hbm,
                  tab_s, acc_s, srcv2, dstv2, rowsv2, csv2,
                  gsem, isem0, isem1, ssem0, ssem1):
        cid = lax.axis_index("c")
        sid = lax.axis_index("s")
        wid = sid * NC + cid
        rows_per = NPAD // NS
        r0 = sid * rows_per
        pltpu.sync_copy(zero_hbm.at[pl.ds(r0, rows_per)],
                        acc_s.at[pl.ds(r0, rows_per)])
        plsc.subcore_barrier()
        base0 = wid * EPW
        isem = [isem0, isem1]
        ssem = [ssem0, ssem1]

        def in_copies(k, b):
            base = base0 + k * CHUNK
            cps = [(src_hbm.at[pl.ds(base, CHUNK)], srcv2.at[b]),
                   (dst_hbm.at[pl.ds(base, CHUNK)], dstv2.at[b])]
            if rotate:
                cps.append((cs_hbm.at[:, pl.ds(base, CHUNK)], csv2.at[b]))
            return cps

        def in_start(k, b):
            for s, d in in_copies(k, b):
                pltpu.async_copy(s, d, isem[b])

        def in_wait(k, b):
            for s, d in in_copies(k, b):
                pltpu.make_async_copy(s, d, isem[b]).wait()

        def gath(b):
            pltpu.async_copy(table_hbm.at[srcv2.at[b]], rowsv2.at[b],
                             gsem).wait()

        rows_out = rowsv2

        def sc_start(b):
            pltpu.async_copy(rows_out.at[b], acc_s.at[dstv2.at[b]], ssem[b],
                             add=True)

        def sc_wait(b):
            pltpu.make_async_copy(rows_out.at[b], acc_s.at[dstv2.at[b]],
                                  ssem[b]).wait()

        lane = lax.iota(jnp.int32, 16)
        lo = lane < 8
        crow = jnp.where(lo, 0, 2)
        srow = jnp.where(lo, 1, 3)

        def rot_chunk(b):
            rb = rowsv2.at[b]
            cb = csv2.at[b]

            @plsc.parallel_loop(0, CHUNK, unroll=16)
            def _(e):
                a = rb[e, 0:16]
                bb = rb[e, 16:32]
                ecol = jnp.full((16,), e, jnp.int32)
                c = plsc.load_gather(cb, [crow, ecol])
                s = plsc.load_gather(cb, [srow, ecol])
                rb[e, 0:16] = c * a - s * bb
                rb[e, 16:32] = s * a + c * bb

        def process(k, b, pk, first=False):
            in_wait(k, b)
            if not first:
                sc_wait(1 - b)
            if pk is not None:
                in_start(pk, 1 - b)
            gath(b)
            if rotate:
                rot_chunk(b)
            sc_start(b)

        in_start(0, 0)
        process(0, 0, 1, first=True)

        def pair(p, carry):
            ka = 2 * p + 1
            process(ka, 1, ka + 1)
            process(ka + 1, 0, ka + 2)
            return carry

        if NCHUNK % 2 == 1:
            lax.fori_loop(0, (NCHUNK - 3) // 2, pair, 0)
            process(NCHUNK - 2, 1, NCHUNK - 1)
            process(NCHUNK - 1, 0, None)
            sc_wait(0)
        else:
            lax.fori_loop(0, (NCHUNK - 2) // 2, pair, 0)
            process(NCHUNK - 1, 1, None)
            sc_wait(1)
        plsc.subcore_barrier()
        pltpu.sync_copy(acc_s.at[pl.ds(r0, rows_per)],
                        out_hbm.at[cid, pl.ds(r0, rows_per)])

    if rotate:
        body = body_impl
    else:
        def body(table_hbm, src_hbm, dst_hbm, zero_hbm, out_hbm, *rest):
            body_impl(table_hbm, src_hbm, dst_hbm, None, zero_hbm, out_hbm,
                      *rest)

    return pl.kernel(
        body,
        out_type=jax.ShapeDtypeStruct((NC, NPAD, PADW), jnp.float32),
        mesh=mesh,
        scratch_types=[
            pltpu.VMEM_SHARED((NPAD, PADW), jnp.float32),
            pltpu.VMEM_SHARED((NPAD, PADW), jnp.float32),
            pltpu.VMEM((2, CHUNK), jnp.int32),
            pltpu.VMEM((2, CHUNK), jnp.int32),
            pltpu.VMEM((2, CHUNK, PADW), jnp.float32),
            pltpu.VMEM((2, 8, CHUNK), jnp.float32),
            pltpu.SemaphoreType.DMA,
            pltpu.SemaphoreType.DMA,
            pltpu.SemaphoreType.DMA,
            pltpu.SemaphoreType.DMA,
            pltpu.SemaphoreType.DMA,
        ],
        compiler_params=pltpu.CompilerParams(use_tc_tiling_on_sc=False,
                                             needs_layout_passes=False),
        interpret=interpret,
    )


_conv_plain = _make_conv(False)
_conv_rot = _make_conv(True)


def _nonlin(x):
    a = x[:, 0:16]
    b = x[:, 16:32]
    r = jnp.sqrt(a * a + b * b + 1e-6)
    sc = jnp.maximum(r - 0.1, 0.0) / r
    rest = jnp.maximum(x[:, 32:], 0.0)
    return jnp.concatenate([a * sc, b * sc, rest], axis=1)


def _combine0_body(acc_ref, hs_ref, t1_ref, deg_ref):
    a = acc_ref[0] + acc_ref[1]
    deg = jnp.maximum(a[:, 40:41], 1.0)
    h1 = hs_ref[...] + a / deg
    col = lax.broadcasted_iota(jnp.int32, h1.shape, 1)
    t1_ref[...] = _nonlin(jnp.where(col < 40, h1, 0.0))
    deg_ref[...] = deg


def _combine0(acc, hs0):
    B = 2000
    return pl.pallas_call(
        _combine0_body,
        grid=(N // B,),
        in_specs=[pl.BlockSpec((NC, B, PADW), lambda i: (0, i, 0)),
                  pl.BlockSpec((B, PADW), lambda i: (i, 0))],
        out_specs=[pl.BlockSpec((B, PADW), lambda i: (i, 0)),
                   pl.BlockSpec((B, 1), lambda i: (i, 0))],
        out_shape=[jax.ShapeDtypeStruct((N, PADW), jnp.float32),
                   jax.ShapeDtypeStruct((N, 1), jnp.float32)],
    )(acc, hs0)


def _combine1_body(acc_ref, h_ref, deg_ref, ws_ref, wn_ref, out_ref):
    agg = (acc_ref[0] + acc_ref[1]) / deg_ref[...]
    h2 = (jnp.dot(h_ref[...], ws_ref[...], preferred_element_type=jnp.float32)
          + jnp.dot(agg, wn_ref[...], preferred_element_type=jnp.float32))
    col = lax.broadcasted_iota(jnp.int32, h2.shape, 1)
    out_ref[...] = _nonlin(jnp.where(col < 40, h2, 0.0))


def _combine1(acc, h, deg, wsp, wnp):
    B = 2000
    return pl.pallas_call(
        _combine1_body,
        grid=(N // B,),
        in_specs=[pl.BlockSpec((NC, B, PADW), lambda i: (0, i, 0)),
                  pl.BlockSpec((B, PADW), lambda i: (i, 0)),
                  pl.BlockSpec((B, 1), lambda i: (i, 0)),
                  pl.BlockSpec((PADW, PADW), lambda i: (0, 0)),
                  pl.BlockSpec((PADW, PADW), lambda i: (0, 0))],
        out_specs=pl.BlockSpec((B, PADW), lambda i: (i, 0)),
        out_shape=jax.ShapeDtypeStruct((N, PADW), jnp.float32),
    )(acc, h, deg, wsp, wnp)


def _combine2_body(acc_ref, h_ref, deg_ref, ws_ref, wn_ref, out_ref):
    agg = (acc_ref[0] + acc_ref[1]) / deg_ref[...]
    out = (jnp.dot(h_ref[...], ws_ref[...], preferred_element_type=jnp.float32)
           + jnp.dot(agg, wn_ref[...], preferred_element_type=jnp.float32))
    out_ref[...] = jnp.maximum(out, 0.0)


def _combine2(acc, h, deg, wsp, wnp):
    B = 2000
    return pl.pallas_call(
        _combine2_body,
        grid=(N // B,),
        in_specs=[pl.BlockSpec((NC, B, PADW), lambda i: (0, i, 0)),
                  pl.BlockSpec((B, PADW), lambda i: (i, 0)),
                  pl.BlockSpec((B, 1), lambda i: (i, 0)),
                  pl.BlockSpec((PADW, 128), lambda i: (0, 0)),
                  pl.BlockSpec((PADW, 128), lambda i: (0, 0))],
        out_specs=pl.BlockSpec((B, 128), lambda i: (i, 0)),
        out_shape=jax.ShapeDtypeStruct((N, 128), jnp.float32),
    )(acc, h, deg, wsp, wnp)


def kernel(pos, x, theta, Ws0, Wn0, Ws1, Wn1, Ws2, Wn2, edge_index):
    P = jnp.asarray(_PERM)
    h = jnp.concatenate([pos, x], axis=1)
    src = edge_index[0]
    dst = edge_index[1]
    wn0p = jnp.pad(Wn0[:, P], ((0, 0), (0, 8)))
    ws0p = jnp.pad(Ws0[:, P], ((0, 0), (0, 8)))
    ws1p = jnp.pad(Ws1[P][:, P], ((0, 8), (0, 8)))
    wn1p = jnp.pad(Wn1[P][:, P], ((0, 8), (0, 8)))
    ws2p = jnp.pad(Ws2[P], ((0, 8), (0, 0)))
    wn2p = jnp.pad(Wn2[P], ((0, 8), (0, 0)))
    zero = jnp.zeros((NPAD, PADW), jnp.float32)

    def padn(t):
        return jnp.pad(t, ((0, NPAD - N), (0, 0)))

    y0, hs0 = _prep_nodes(h, wn0p, ws0p)
    acc = _conv_plain(padn(y0), src, dst, zero)
    cs = _prep_trig(theta.reshape(1, E))
    t1, deg = _combine0(acc, hs0)
    acc = _conv_rot(padn(t1), src, dst, cs, zero)
    t2 = _combine1(acc, t1, deg, ws1p, wn1p)
    acc = _conv_rot(padn(t2), src, dst, cs, zero)
    return _combine2(acc, t2, deg, ws2p, wn2p)


# speedup vs baseline: 1.1332x; 1.1332x over previous
import functools

import numpy as np
import jax
import jax.numpy as jnp
from jax import lax
from jax.experimental import pallas as pl
from jax.experimental.pallas import tpu as pltpu
from jax.experimental.pallas import tpu_sc as plsc

N = 10000
NPAD = 10240
E = 320000
PADW = 48
NC, NS = 2, 16
NW = NC * NS
EPW = E // NW
CHUNK = 200
NCHUNK = EPW // CHUNK

_pa = np.array([8 + 2 * i for i in range(8)] + [24 + 2 * i for i in range(8)],
               np.int32)
_PERM = np.concatenate([_pa, _pa + 1, np.arange(8, dtype=np.int32)])


def _prep_nodes_body(h_ref, wn_ref, ws_ref, y0_ref, hs_ref):
    h = h_ref[...]
    y0 = jnp.dot(h, wn_ref[...], preferred_element_type=jnp.float32)
    col = lax.broadcasted_iota(jnp.int32, y0.shape, 1)
    y0_ref[...] = jnp.where(col == 40, 1.0, y0)
    hs_ref[...] = jnp.dot(h, ws_ref[...], preferred_element_type=jnp.float32)


def _prep_nodes(h, wn0p, ws0p):
    B = 2000
    return pl.pallas_call(
        _prep_nodes_body,
        grid=(N // B,),
        in_specs=[pl.BlockSpec((B, 128), lambda i: (i, 0)),
                  pl.BlockSpec((128, PADW), lambda i: (0, 0)),
                  pl.BlockSpec((128, PADW), lambda i: (0, 0))],
        out_specs=[pl.BlockSpec((B, PADW), lambda i: (i, 0)),
                   pl.BlockSpec((B, PADW), lambda i: (i, 0))],
        out_shape=[jax.ShapeDtypeStruct((N, PADW), jnp.float32),
                   jax.ShapeDtypeStruct((N, PADW), jnp.float32)],
    )(h, wn0p, ws0p)


def _prep_trig_body(th_ref, cs_ref):
    t = th_ref[...]
    c1 = jnp.cos(t)
    s1 = jnp.sin(t)
    c2 = c1 * c1 - s1 * s1
    s2 = 2.0 * c1 * s1
    row = lax.broadcasted_iota(jnp.int32, (8, t.shape[1]), 0)
    cs_ref[...] = jnp.where(row == 0, c1,
                            jnp.where(row == 1, s1,
                                      jnp.where(row == 2, c2,
                                                jnp.where(row == 3, s2, 0.0))))


def _prep_trig(theta2d):
    B = 16000
    return pl.pallas_call(
        _prep_trig_body,
        grid=(E // B,),
        in_specs=[pl.BlockSpec((1, B), lambda i: (0, i))],
        out_specs=pl.BlockSpec((8, B), lambda i: (0, i)),
        out_shape=jax.ShapeDtypeStruct((8, E), jnp.float32),
    )(theta2d)


def _make_conv(rotate, w=PADW, interpret=False):
    mesh = plsc.VectorSubcoreMesh(core_axis_name="c", subcore_axis_name="s",
                                  num_cores=NC, num_subcores=NS)

    def body_impl(table_hbm, src_hbm, dst_hbm, cs_hbm, zero_hbm, out_hbm,
                  tab_s, acc_s, srcv2, dstv2, rowsv2, csv2,
                  gsem, isem0, isem1, ssem0, ssem1):
        cid = lax.axis_index("c")
        sid = lax.axis_index("s")
        wid = sid * NC + cid
        rows_per = NPAD // NS
        r0 = sid * rows_per
        pltpu.sync_copy(table_hbm.at[pl.ds(r0, rows_per)],
                        tab_s.at[pl.ds(r0, rows_per)])
        pltpu.sync_copy(zero_hbm.at[pl.ds(r0, rows_per)],
                        acc_s.at[pl.ds(r0, rows_per)])
        plsc.subcore_barrier()
        base0 = wid * EPW
        isem = [isem0, isem1]
        ssem = [ssem0, ssem1]

        def in_copies(k, b):
            base = base0 + k * CHUNK
            cps = [(src_hbm.at[pl.ds(base, CHUNK)], srcv2.at[b]),
                   (dst_hbm.at[pl.ds(base, CHUNK)], dstv2.at[b])]
            if rotate:
                cps.append((cs_hbm.at[:, pl.ds(base, CHUNK)], csv2.at[b]))
            return cps

        def in_start(k, b):
            for s, d in in_copies(k, b):
                pltpu.async_copy(s, d, isem[b])

        def in_wait(k, b):
            for s, d in in_copies(k, b):
                pltpu.make_async_copy(s, d, isem[b]).wait()

        def gath(b):
            pltpu.async_copy(tab_s.at[srcv2.at[b]], rowsv2.at[b], gsem).wait()

        rows_out = rowsv2

        def sc_start(b):
            pltpu.async_copy(rows_out.at[b], acc_s.at[dstv2.at[b]], ssem[b],
                             add=True)

        def sc_wait(b):
            pltpu.make_async_copy(rows_out.at[b], acc_s.at[dstv2.at[b]],
                                  ssem[b]).wait()

        lane = lax.iota(jnp.int32, 16)
        lo = lane < 8
        crow = jnp.where(lo, 0, 2)
        srow = jnp.where(lo, 1, 3)

        def rot_chunk(b):
            rb = rowsv2.at[b]
            cb = csv2.at[b]

            @plsc.parallel_loop(0, CHUNK, unroll=16)
            def _(e):
                a = rb[e, 0:16]
                bb = rb[e, 16:32]
                ecol = jnp.full((16,), e, jnp.int32)
                c = plsc.load_gather(cb, [crow, ecol])
                s = plsc.load_gather(cb, [srow, ecol])
                rb[e, 0:16] = c * a - s * bb
                rb[e, 16:32] = s * a + c * bb

        def process(k, b, pk, first=False):
            in_wait(k, b)
            if not first:
                sc_wait(1 - b)
            if pk is not None:
                in_start(pk, 1 - b)
            gath(b)
            if rotate:
                rot_chunk(b)
            sc_start(b)

        in_start(0, 0)
        process(0, 0, 1, first=True)

        def pair(p, carry):
            ka = 2 * p + 1
            process(ka, 1, ka + 1)
            process(ka + 1, 0, ka + 2)
            return carry

        if NCHUNK % 2 == 1:
            lax.fori_loop(0, (NCHUNK - 3) // 2, pair, 0)
            process(NCHUNK - 2, 1, NCHUNK - 1)
            process(NCHUNK - 1, 0, None)
            sc_wait(0)
        else:
            lax.fori_loop(0, (NCHUNK - 2) // 2, pair, 0)
            process(NCHUNK - 1, 1, None)
            sc_wait(1)
        plsc.subcore_barrier()
        pltpu.sync_copy(acc_s.at[pl.ds(r0, rows_per)],
                        out_hbm.at[cid, pl.ds(r0, rows_per)])

    if rotate:
        body = body_impl
    else:
        def body(table_hbm, src_hbm, dst_hbm, zero_hbm, out_hbm, *rest):
            body_impl(table_hbm, src_hbm, dst_hbm, None, zero_hbm, out_hbm,
                      *rest)

    return pl.kernel(
        body,
        out_type=jax.ShapeDtypeStruct((NC, NPAD, w), jnp.float32),
        mesh=mesh,
        scratch_types=[
            pltpu.VMEM_SHARED((NPAD, w), jnp.float32),
            pltpu.VMEM_SHARED((NPAD, w), jnp.float32),
            pltpu.VMEM((2, CHUNK), jnp.int32),
            pltpu.VMEM((2, CHUNK), jnp.int32),
            pltpu.VMEM((2, CHUNK, w), jnp.float32),
            pltpu.VMEM((2, 8, CHUNK), jnp.float32),
            pltpu.SemaphoreType.DMA,
            pltpu.SemaphoreType.DMA,
            pltpu.SemaphoreType.DMA,
            pltpu.SemaphoreType.DMA,
            pltpu.SemaphoreType.DMA,
        ],
        compiler_params=pltpu.CompilerParams(use_tc_tiling_on_sc=False,
                                             needs_layout_passes=False),
        interpret=interpret,
    )


_conv_plain = _make_conv(False, 48)
_conv_rot = _make_conv(True, 40)


def _nonlin(x):
    a = x[:, 0:16]
    b = x[:, 16:32]
    r = jnp.sqrt(a * a + b * b + 1e-6)
    sc = jnp.maximum(r - 0.1, 0.0) / r
    rest = jnp.maximum(x[:, 32:], 0.0)
    return jnp.concatenate([a * sc, b * sc, rest], axis=1)


def _combine0_body(acc_ref, hs_ref, t1_ref, deg_ref):
    a = acc_ref[0] + acc_ref[1]
    deg = jnp.maximum(a[:, 40:41], 1.0)
    h1 = hs_ref[:, 0:40] + a[:, 0:40] / deg
    t1_ref[...] = _nonlin(h1)
    deg_ref[...] = deg


def _combine0(acc, hs0):
    B = 2000
    return pl.pallas_call(
        _combine0_body,
        grid=(N // B,),
        in_specs=[pl.BlockSpec((NC, B, 48), lambda i: (0, i, 0)),
                  pl.BlockSpec((B, 48), lambda i: (i, 0))],
        out_specs=[pl.BlockSpec((B, 40), lambda i: (i, 0)),
                   pl.BlockSpec((B, 1), lambda i: (i, 0))],
        out_shape=[jax.ShapeDtypeStruct((N, 40), jnp.float32),
                   jax.ShapeDtypeStruct((N, 1), jnp.float32)],
    )(acc, hs0)


def _combine1_body(acc_ref, h_ref, deg_ref, ws_ref, wn_ref, out_ref):
    agg = (acc_ref[0] + acc_ref[1]) / deg_ref[...]
    h2 = (jnp.dot(h_ref[...], ws_ref[...], preferred_element_type=jnp.float32)
          + jnp.dot(agg, wn_ref[...], preferred_element_type=jnp.float32))
    out_ref[...] = _nonlin(h2)


def _combine1(acc, h, deg, wsp, wnp):
    B = 2000
    return pl.pallas_call(
        _combine1_body,
        grid=(N // B,),
        in_specs=[pl.BlockSpec((NC, B, 40), lambda i: (0, i, 0)),
                  pl.BlockSpec((B, 40), lambda i: (i, 0)),
                  pl.BlockSpec((B, 1), lambda i: (i, 0)),
                  pl.BlockSpec((40, 40), lambda i: (0, 0)),
                  pl.BlockSpec((40, 40), lambda i: (0, 0))],
        out_specs=pl.BlockSpec((B, 40), lambda i: (i, 0)),
        out_shape=jax.ShapeDtypeStruct((N, 40), jnp.float32),
    )(acc, h, deg, wsp, wnp)


def _combine2_body(acc_ref, h_ref, deg_ref, ws_ref, wn_ref, out_ref):
    agg = (acc_ref[0] + acc_ref[1]) / deg_ref[...]
    out = (jnp.dot(h_ref[...], ws_ref[...], preferred_element_type=jnp.float32)
           + jnp.dot(agg, wn_ref[...], preferred_element_type=jnp.float32))
    out_ref[...] = jnp.maximum(out, 0.0)


def _combine2(acc, h, deg, wsp, wnp):
    B = 2000
    return pl.pallas_call(
        _combine2_body,
        grid=(N // B,),
        in_specs=[pl.BlockSpec((NC, B, 40), lambda i: (0, i, 0)),
                  pl.BlockSpec((B, 40), lambda i: (i, 0)),
                  pl.BlockSpec((B, 1), lambda i: (i, 0)),
                  pl.BlockSpec((40, 128), lambda i: (0, 0)),
                  pl.BlockSpec((40, 128), lambda i: (0, 0))],
        out_specs=pl.BlockSpec((B, 128), lambda i: (i, 0)),
        out_shape=jax.ShapeDtypeStruct((N, 128), jnp.float32),
    )(acc, h, deg, wsp, wnp)


def kernel(pos, x, theta, Ws0, Wn0, Ws1, Wn1, Ws2, Wn2, edge_index):
    P = jnp.asarray(_PERM)
    h = jnp.concatenate([pos, x], axis=1)
    src = edge_index[0]
    dst = edge_index[1]
    wn0p = jnp.pad(Wn0[:, P], ((0, 0), (0, 8)))
    ws0p = jnp.pad(Ws0[:, P], ((0, 0), (0, 8)))
    ws1p = Ws1[P][:, P]
    wn1p = Wn1[P][:, P]
    ws2p = Ws2[P]
    wn2p = Wn2[P]
    zero48 = jnp.zeros((NPAD, 48), jnp.float32)
    zero40 = jnp.zeros((NPAD, 40), jnp.float32)

    def padn(t):
        return jnp.pad(t, ((0, NPAD - N), (0, 0)))

    y0, hs0 = _prep_nodes(h, wn0p, ws0p)
    acc = _conv_plain(padn(y0), src, dst, zero48)
    cs = _prep_trig(theta.reshape(1, E))
    t1, deg = _combine0(acc, hs0)
    acc = _conv_rot(padn(t1), src, dst, cs, zero40)
    t2 = _combine1(acc, t1, deg, ws1p, wn1p)
    acc = _conv_rot(padn(t2), src, dst, cs, zero40)
    return _combine2(acc, t2, deg, ws2p, wn2p)
